# baseline (speedup 1.0000x reference)
import functools
import math

import jax
import jax.numpy as jnp
from jax import lax
from jax.experimental import pallas as pl
from jax.experimental.pallas import tpu as pltpu
from jax.experimental.pallas import tpu_sc as plsc

_NC = 2
_NS = 16
_NW = _NC * _NS


def _emb_kernel(T, NB, D, scale):
    UB = 1024
    JU = NB // UB
    n_units = T * JU
    u_per_w = n_units // _NW
    EB = D // 8
    INNER = UB * 8 * 128 // 128
    mesh = plsc.VectorSubcoreMesh(core_axis_name="c", subcore_axis_name="s")

    @functools.partial(
        pl.kernel,
        mesh=mesh,
        compiler_params=pltpu.CompilerParams(
            use_tc_tiling_on_sc=False, needs_layout_passes=False),
        out_type=jax.ShapeDtypeStruct((T, EB, NB * 8), jnp.float32),
        scratch_types=[
            pltpu.VMEM((UB,), jnp.int32),
            pltpu.VMEM((UB, D), jnp.float32),
            pltpu.VMEM((EB, UB * 8), jnp.float32),
            pltpu.SemaphoreType.DMA,
        ],
    )
    def emb_k(tok_hbm, table_hbm, out_hbm, idx_v, rows_v, tbuf_v, sem):
        wid = lax.axis_index("s") * _NC + lax.axis_index("c")

        def unit_body(ul, carry):
            u = wid * u_per_w + ul
            t = u // JU
            j = u % JU
            toff = pl.multiple_of(t * NB + j * UB, 8)
            pltpu.sync_copy(tok_hbm.at[pl.ds(toff, UB)], idx_v)
            pltpu.async_copy(table_hbm.at[idx_v], rows_v, sem).wait()

            def bb_body(bb, c1):
                def bl_body(bl, c2):
                    r = bb * 128 + bl
                    lane = lax.iota(jnp.int32, 16)
                    eight = jnp.full((16,), 8, jnp.int32)
                    eb_idx = lane // eight
                    es128 = (lane % eight) * jnp.full((16,), 128, jnp.int32)
                    inner = es128 + jnp.full((16,), bb * 1024, jnp.int32) \
                        + jnp.full((16,), bl, jnp.int32)
                    v0 = rows_v[r, 0:16] * scale
                    v1 = rows_v[r, 16:32] * scale
                    plsc.store_scatter(tbuf_v, [eb_idx, inner], v0)
                    plsc.store_scatter(
                        tbuf_v, [eb_idx + jnp.full((16,), 2, jnp.int32),
                                 inner], v1)
                    return c2
                lax.fori_loop(0, 128, bl_body, 0)
                return c1
            lax.fori_loop(0, 8, bb_body, 0)

            pltpu.sync_copy(tbuf_v, out_hbm.at[t, :, pl.ds(j * INNER, INNER)])
            return carry

        lax.fori_loop(0, u_per_w, unit_body, 0)

    return emb_k


def kernel(tokens, table):
    B, T = tokens.shape
    V, D = table.shape
    scale = math.sqrt(D)

    tok_t = tokens.T.reshape(B * T)
    emb_k = _emb_kernel(T, B, D, scale)
    out5 = emb_k(tok_t, table)

    out = (out5.reshape(T, D // 8, B // 128, 8, 128)
               .transpose(2, 4, 0, 1, 3)
               .reshape(B, T, D))
    return out


# speedup vs baseline: 1.0964x; 1.0964x over previous
import functools
import math

import jax
import jax.numpy as jnp
from jax import lax
from jax.experimental import pallas as pl
from jax.experimental.pallas import tpu as pltpu
from jax.experimental.pallas import tpu_sc as plsc

_NC = 2
_NS = 16
_NW = _NC * _NS


def _emb_kernel(T, NB, D, scale):
    UB = 1024
    JU = NB // UB
    n_units = T * JU
    u_per_w = n_units // _NW
    EB = D // 8
    INNER = UB * 8 * 128 // 128
    mesh = plsc.VectorSubcoreMesh(core_axis_name="c", subcore_axis_name="s")

    @functools.partial(
        pl.kernel,
        mesh=mesh,
        compiler_params=pltpu.CompilerParams(
            use_tc_tiling_on_sc=False, needs_layout_passes=False),
        out_type=jax.ShapeDtypeStruct((T, EB, NB * 8), jnp.float32),
        scratch_types=[
            pltpu.VMEM((UB,), jnp.int32),
            pltpu.VMEM((UB, D), jnp.float32),
            pltpu.VMEM((EB, UB * 8), jnp.float32),
            pltpu.SemaphoreType.DMA,
        ],
    )
    def emb_k(tok_hbm, table_hbm, out_hbm, idx_v, rows_v, tbuf_v, sem):
        wid = lax.axis_index("s") * _NC + lax.axis_index("c")

        def unit_body(ul, carry):
            u = wid * u_per_w + ul
            t = u // JU
            j = u % JU
            toff = pl.multiple_of(t * NB + j * UB, 8)
            pltpu.sync_copy(tok_hbm.at[pl.ds(toff, UB)], idx_v)
            pltpu.async_copy(table_hbm.at[idx_v], rows_v, sem).wait()

            lane = lax.iota(jnp.int32, 16)
            eight = jnp.full((16,), 8, jnp.int32)
            eb_lo = lane // eight
            eb_hi = eb_lo + jnp.full((16,), 2, jnp.int32)
            es128 = (lane % eight) * jnp.full((16,), 128, jnp.int32)

            @plsc.parallel_loop(0, UB, unroll=8)
            def trans_body(r):
                bs = ((r >> 7) << 10) + (r & 127)
                inner = es128 + jnp.full((16,), bs, jnp.int32)
                v0 = rows_v[r, 0:16] * scale
                v1 = rows_v[r, 16:32] * scale
                plsc.store_scatter(tbuf_v, [eb_lo, inner], v0)
                plsc.store_scatter(tbuf_v, [eb_hi, inner], v1)

            pltpu.sync_copy(tbuf_v, out_hbm.at[t, :, pl.ds(j * INNER, INNER)])
            return carry

        lax.fori_loop(0, u_per_w, unit_body, 0)

    return emb_k


def kernel(tokens, table):
    B, T = tokens.shape
    V, D = table.shape
    scale = math.sqrt(D)

    tok_t = tokens.T.reshape(B * T)
    emb_k = _emb_kernel(T, B, D, scale)
    out5 = emb_k(tok_t, table)

    out = (out5.reshape(T, D // 8, B // 128, 8, 128)
               .transpose(2, 4, 0, 1, 3)
               .reshape(B, T, D))
    return out


# speedup vs baseline: 1.6694x; 1.5227x over previous
import functools
import math

import jax
import jax.numpy as jnp
from jax import lax
from jax.experimental import pallas as pl
from jax.experimental.pallas import tpu as pltpu
from jax.experimental.pallas import tpu_sc as plsc

_NC = 2
_NS = 16
_NW = _NC * _NS


def _emb_kernel(T, NB, D, scale):
    UB = 1024
    JU = NB // UB
    n_units = T * JU
    u_per_w = n_units // _NW
    EB = D // 8
    ROWS = EB * (UB // 128) * 8
    PAD = 129
    mesh = plsc.VectorSubcoreMesh(core_axis_name="c", subcore_axis_name="s")

    @functools.partial(
        pl.kernel,
        mesh=mesh,
        compiler_params=pltpu.CompilerParams(
            use_tc_tiling_on_sc=False, needs_layout_passes=False),
        out_type=jax.ShapeDtypeStruct((T, EB * (NB // 128) * 8, 128),
                                      jnp.float32),
        scratch_types=[
            pltpu.VMEM((UB,), jnp.int32),
            pltpu.VMEM((UB, D), jnp.float32),
            pltpu.VMEM((ROWS, PAD), jnp.float32),
            pltpu.SemaphoreType.DMA,
        ],
    )
    def emb_k(tok_hbm, table_hbm, out_hbm, idx_v, rows_v, tbuf_v, sem):
        wid = lax.axis_index("s") * _NC + lax.axis_index("c")
        lane = lax.iota(jnp.int32, 16)
        eight = jnp.full((16,), 8, jnp.int32)
        base_row = (lane // eight) * jnp.full((16,), 64, jnp.int32) \
            + (lane % eight)
        hi_off = jnp.full((16,), 128, jnp.int32)

        def unit_body(ul, carry):
            u = wid * u_per_w + ul
            t = u // JU
            j = u % JU
            toff = pl.multiple_of(t * NB + j * UB, 8)
            pltpu.sync_copy(tok_hbm.at[pl.ds(toff, UB)], idx_v)
            pltpu.async_copy(table_hbm.at[idx_v], rows_v, sem).wait()

            @plsc.parallel_loop(0, UB, unroll=8)
            def trans_body(r):
                row0 = base_row + jnp.full((16,), (r >> 7) << 3, jnp.int32)
                col = jnp.full((16,), r & 127, jnp.int32)
                v0 = rows_v[r, 0:16] * scale
                v1 = rows_v[r, 16:32] * scale
                plsc.store_scatter(tbuf_v, [row0, col], v0)
                plsc.store_scatter(tbuf_v, [row0 + hi_off, col], v1)

            for eb in range(EB):
                pltpu.sync_copy(
                    tbuf_v.at[pl.ds(eb * 64, 64), pl.ds(0, 128)],
                    out_hbm.at[t, pl.ds(eb * (NB // 128) * 8 + j * 64, 64), :])
            return carry

        lax.fori_loop(0, u_per_w, unit_body, 0)

    return emb_k


def kernel(tokens, table):
    B, T = tokens.shape
    V, D = table.shape
    scale = math.sqrt(D)

    tok_t = tokens.T.reshape(B * T)
    emb_k = _emb_kernel(T, B, D, scale)
    out5 = emb_k(tok_t, table)

    out = (out5.reshape(T, D // 8, B // 128, 8, 128)
               .transpose(2, 4, 0, 1, 3)
               .reshape(B, T, D))
    return out
